# baseline (speedup 1.0000x reference)
import jax
import jax.numpy as jnp
from jax.experimental import pallas as pl

DEPTH = 12


def _se(h, W1, W2):
    y = jax.nn.sigmoid(jax.nn.relu(h @ W1.T) @ W2.T)
    return h * y


def _bn(h, g, b, eps=1e-5):
    mu = jnp.mean(h, axis=0)
    var = jnp.var(h, axis=0)
    return g * (h - mu) * jax.lax.rsqrt(var + eps) + b


def _seg_mean(vals, idx, n):
    s = jax.ops.segment_sum(vals, idx, num_segments=n)
    cnt = jax.ops.segment_sum(jnp.ones((vals.shape[0], 1), vals.dtype), idx, num_segments=n)
    return s / jnp.maximum(cnt, 1.0)


def _copy_kernel(x_ref, o_ref):
    o_ref[...] = x_ref[...]


def kernel(x, edge_index, edge_attr, v_lin0_W, v_lin0_b, v1_W, v1_b, v2_W, v2_b, v3_W, v3_b, v4_W, v4_b, vbn_g, vbn_b, e_lin0_W, e_lin0_b, e0_W, e0_b, ebn_g, ebn_b, se_W1, se_W2):
    n = x.shape[0]
    src = edge_index[0]
    dst = edge_index[1]
    h = jax.nn.silu(x @ v_lin0_W.T + v_lin0_b)
    w = jax.nn.silu(edge_attr @ e_lin0_W.T + e_lin0_b)
    for i in range(DEPTH):
        x0 = h
        x1 = _se(x0 @ v1_W[i].T + v1_b[i], se_W1, se_W2)
        x2 = _se(x0 @ v2_W[i].T + v2_b[i], se_W1, se_W2)
        x3 = _se(x0 @ v3_W[i].T + v3_b[i], se_W1, se_W2)
        x4 = _se(x0 @ v4_W[i].T + v4_b[i], se_W1, se_W2)
        w0 = w
        w1 = w0 @ e0_W[i].T + e0_b[i]
        w2 = jax.nn.sigmoid(w0)
        agg = _seg_mean(w2 * x2[dst], src, n)
        h = x0 + jax.nn.silu(_bn(x1 + agg, vbn_g[i], vbn_b[i]))
        w = w0 + jax.nn.silu(_bn(w1 + x3[src] + x4[dst], ebn_g[i], ebn_b[i]))
    w4 = w.reshape(80000, 128)
    w4 = pl.pallas_call(
        _copy_kernel,
        grid=(10,),
        in_specs=[pl.BlockSpec((8000, 128), lambda i: (i, 0))],
        out_specs=pl.BlockSpec((8000, 128), lambda i: (i, 0)),
        out_shape=jax.ShapeDtypeStruct((80000, 128), w.dtype),
    )(w4)
    return w4.reshape(320000, 32)


# speedup vs baseline: 2.6385x; 2.6385x over previous
import functools

import jax
import jax.numpy as jnp
from jax import lax
from jax.experimental import pallas as pl
from jax.experimental.pallas import tpu as pltpu
from jax.experimental.pallas import tpu_sc as plsc

DEPTH = 12
UNITS = 32
N_NODES = 10000
N_EDGES = 320000

NC = 2
NS = 16
NW = NC * NS
EPW = N_EDGES // NW
G = 80
NCH = EPW // G
NRD = 10
ROWS_RD = N_NODES // NRD
CNT_W = 16


def _sigmoid(v):
    return 1.0 / (1.0 + jnp.exp(-v))


def _sc_edge_body(src_hbm, dst_hbm, w_hbm, x2_hbm, x3_hbm, x4_hbm,
                  g_hbm, aggp_hbm,
                  srcv, dstv, wv, r2, r3, r4, cbuf, gbuf, ob,
                  shared_agg, sem0, sem1, sem2):
    cid = lax.axis_index("c")
    sid = lax.axis_index("s")
    wid = sid * NC + cid

    @pl.when(sid < NRD)
    def _zero():
        def _zrow(r, _):
            z = jnp.zeros((16,), jnp.float32)
            ob[r, pl.ds(0, 16)] = z
            ob[r, pl.ds(16, 16)] = z
            return _
        lax.fori_loop(0, ROWS_RD, _zrow, None)
        pltpu.sync_copy(ob, shared_agg.at[pl.ds(sid * ROWS_RD, ROWS_RD)])

    plsc.subcore_barrier()

    base = wid * EPW

    def _group(gi, _):
        row0 = base + gi * G
        pltpu.sync_copy(src_hbm.at[pl.ds(row0, G)], srcv)
        pltpu.sync_copy(dst_hbm.at[pl.ds(row0, G)], dstv)
        c2 = pltpu.async_copy(x2_hbm.at[dstv], r2, sem0)
        c3 = pltpu.async_copy(x3_hbm.at[srcv], r3, sem1)
        c4 = pltpu.async_copy(x4_hbm.at[dstv], r4, sem2)
        pltpu.sync_copy(w_hbm.at[pl.ds(row0, G)], wv)
        c2.wait()
        c3.wait()
        c4.wait()

        def _row(r, _):
            for h in (0, 16):
                s = _sigmoid(wv[r, pl.ds(h, 16)])
                cbuf[r, pl.ds(h, 16)] = s * r2[r, pl.ds(h, 16)]
                gbuf[r, pl.ds(h, 16)] = r3[r, pl.ds(h, 16)] + r4[r, pl.ds(h, 16)]
            return _
        lax.fori_loop(0, G, _row, None)

        pltpu.sync_copy(gbuf, g_hbm.at[pl.ds(row0, G)])
        pltpu.sync_copy(cbuf, shared_agg.at[srcv], add=True)
        return _

    lax.fori_loop(0, NCH, _group, None)
    plsc.subcore_barrier()

    @pl.when(sid < NRD)
    def _readout():
        pltpu.sync_copy(shared_agg.at[pl.ds(sid * ROWS_RD, ROWS_RD)], ob)
        pltpu.sync_copy(ob, aggp_hbm.at[cid, pl.ds(sid * ROWS_RD, ROWS_RD)])


_sc_edge_kernel = functools.partial(
    pl.kernel,
    _sc_edge_body,
    out_type=[
        jax.ShapeDtypeStruct((N_EDGES, UNITS), jnp.float32),
        jax.ShapeDtypeStruct((NC, N_NODES, UNITS), jnp.float32),
    ],
    mesh=plsc.VectorSubcoreMesh(core_axis_name="c", subcore_axis_name="s"),
    compiler_params=pltpu.CompilerParams(use_tc_tiling_on_sc=False),
    scratch_types=[
        pltpu.VMEM((G,), jnp.int32),
        pltpu.VMEM((G,), jnp.int32),
        pltpu.VMEM((G, UNITS), jnp.float32),
        pltpu.VMEM((G, UNITS), jnp.float32),
        pltpu.VMEM((G, UNITS), jnp.float32),
        pltpu.VMEM((G, UNITS), jnp.float32),
        pltpu.VMEM((G, UNITS), jnp.float32),
        pltpu.VMEM((G, UNITS), jnp.float32),
        pltpu.VMEM((ROWS_RD, UNITS), jnp.float32),
        pltpu.VMEM_SHARED((N_NODES, UNITS), jnp.float32),
        pltpu.SemaphoreType.DMA,
        pltpu.SemaphoreType.DMA,
        pltpu.SemaphoreType.DMA,
    ],
)()


def _sc_cnt_body(src_hbm, cntp_hbm, srcv, ones_buf, ob, shared_cnt):
    cid = lax.axis_index("c")
    sid = lax.axis_index("s")
    wid = sid * NC + cid

    @pl.when(sid < NRD)
    def _zero():
        def _zrow(r, _):
            ob[r, pl.ds(0, 16)] = jnp.zeros((16,), jnp.float32)
            return _
        lax.fori_loop(0, ROWS_RD, _zrow, None)
        pltpu.sync_copy(ob, shared_cnt.at[pl.ds(sid * ROWS_RD, ROWS_RD)])

    def _fill1(r, _):
        ones_buf[r, pl.ds(0, 16)] = jnp.ones((16,), jnp.float32)
        return _
    lax.fori_loop(0, G, _fill1, None)
    plsc.subcore_barrier()

    base = wid * EPW

    def _group(gi, _):
        pltpu.sync_copy(src_hbm.at[pl.ds(base + gi * G, G)], srcv)
        pltpu.sync_copy(ones_buf, shared_cnt.at[srcv], add=True)
        return _

    lax.fori_loop(0, NCH, _group, None)
    plsc.subcore_barrier()

    @pl.when(sid < NRD)
    def _readout():
        pltpu.sync_copy(shared_cnt.at[pl.ds(sid * ROWS_RD, ROWS_RD)], ob)
        pltpu.sync_copy(ob, cntp_hbm.at[cid, pl.ds(sid * ROWS_RD, ROWS_RD)])


_sc_cnt_kernel = functools.partial(
    pl.kernel,
    _sc_cnt_body,
    out_type=[jax.ShapeDtypeStruct((NC, N_NODES, CNT_W), jnp.float32)],
    mesh=plsc.VectorSubcoreMesh(core_axis_name="c", subcore_axis_name="s"),
    compiler_params=pltpu.CompilerParams(use_tc_tiling_on_sc=False),
    scratch_types=[
        pltpu.VMEM((G,), jnp.int32),
        pltpu.VMEM((G, CNT_W), jnp.float32),
        pltpu.VMEM((ROWS_RD, CNT_W), jnp.float32),
        pltpu.VMEM_SHARED((N_NODES, CNT_W), jnp.float32),
    ],
)()


def _se(h, W1, W2):
    y = jax.nn.sigmoid(jax.nn.relu(h @ W1.T) @ W2.T)
    return h * y


def _bn(h, g, b, eps=1e-5):
    mu = jnp.mean(h, axis=0)
    var = jnp.var(h, axis=0)
    return g * (h - mu) * jax.lax.rsqrt(var + eps) + b


def kernel(x, edge_index, edge_attr, v_lin0_W, v_lin0_b, v1_W, v1_b, v2_W, v2_b, v3_W, v3_b, v4_W, v4_b, vbn_g, vbn_b, e_lin0_W, e_lin0_b, e0_W, e0_b, ebn_g, ebn_b, se_W1, se_W2):
    src = edge_index[0]
    dst = edge_index[1]

    cntp = _sc_cnt_kernel(src)[0]
    cnt = jnp.maximum(cntp[0, :, 0] + cntp[1, :, 0], 1.0)[:, None]

    h = jax.nn.silu(x @ v_lin0_W.T + v_lin0_b)
    w = jax.nn.silu(edge_attr @ e_lin0_W.T + e_lin0_b)

    for i in range(DEPTH):
        x0 = h
        x1 = _se(x0 @ v1_W[i].T + v1_b[i], se_W1, se_W2)
        x2 = _se(x0 @ v2_W[i].T + v2_b[i], se_W1, se_W2)
        x3 = _se(x0 @ v3_W[i].T + v3_b[i], se_W1, se_W2)
        x4 = _se(x0 @ v4_W[i].T + v4_b[i], se_W1, se_W2)
        w0 = w

        g, aggp = _sc_edge_kernel(src, dst, w0, x2, x3, x4)
        agg = (aggp[0] + aggp[1]) / cnt

        w1 = w0 @ e0_W[i].T + e0_b[i]
        h = x0 + jax.nn.silu(_bn(x1 + agg, vbn_g[i], vbn_b[i]))
        w = w0 + jax.nn.silu(_bn(w1 + g, ebn_g[i], ebn_b[i]))
    return w
